# baseline (speedup 1.0000x reference)
import functools

import jax
import jax.numpy as jnp
from jax import lax
from jax.experimental import pallas as pl
from jax.experimental.pallas import tpu as pltpu
from jax.experimental.pallas import tpu_sc as plsc

VOCAB = 1000000
HIDDEN = 64
S = 4096
T = 200

NC = 2
NS = 16
NW = NC * NS
S_PER_W = S // NW

TB = 2
NBLK = T // TB
BLK_ROWS = TB * HIDDEN
BLK_LOOKUPS = TB * S_PER_W

_mesh = plsc.VectorSubcoreMesh(core_axis_name="c", subcore_axis_name="s")


@functools.partial(
    pl.kernel,
    mesh=_mesh,
    compiler_params=pltpu.CompilerParams(
        use_tc_tiling_on_sc=False, needs_layout_passes=False
    ),
    out_type=jax.ShapeDtypeStruct((T * HIDDEN, S), jnp.float32),
    scratch_types=[
        pltpu.VMEM((T, S_PER_W), jnp.int32),
        pltpu.VMEM((2, BLK_LOOKUPS, HIDDEN), jnp.float32),
        pltpu.VMEM((2, BLK_ROWS, S_PER_W), jnp.float32),
        pltpu.SemaphoreType.DMA,
        pltpu.SemaphoreType.DMA,
    ],
)
def _embed(ids_hbm, table_hbm, out_hbm, idx_v, g_v, o_v, gsem, ssem):
    wid = lax.axis_index("s") * NC + lax.axis_index("c")
    s0 = pl.multiple_of(wid * S_PER_W, S_PER_W)
    pltpu.sync_copy(ids_hbm.at[:, pl.ds(s0, S_PER_W)], idx_v)

    iota16 = lax.iota(jnp.int32, 16)

    def fire_gathers(tb, b):
        for tj in range(TB):
            pltpu.async_copy(
                table_hbm.at[idx_v.at[tb * TB + tj]],
                g_v.at[b, pl.ds(tj * S_PER_W, S_PER_W)],
                gsem,
            )

    def wait_gathers(b):
        for tj in range(TB):
            pltpu.make_async_copy(
                table_hbm.at[idx_v.at[0]],
                g_v.at[b, pl.ds(tj * S_PER_W, S_PER_W)],
                gsem,
            ).wait()

    def transpose_block(b):
        def per_f(f, _):
            colv = iota16 * 0 + f
            for tj in range(TB):
                for g in range(S_PER_W // 16):
                    rows = tj * S_PER_W + g * 16 + iota16
                    vec = plsc.load_gather(g_v.at[b], [rows, colv])
                    o_v[b, tj * HIDDEN + f, pl.ds(g * 16, 16)] = vec
            return ()
        lax.fori_loop(0, HIDDEN, per_f, ())

    def fire_store(tb, b):
        r0 = pl.multiple_of(tb * BLK_ROWS, 8)
        pltpu.async_copy(
            o_v.at[b],
            out_hbm.at[pl.ds(r0, BLK_ROWS), pl.ds(s0, S_PER_W)],
            ssem,
        )

    def wait_store():
        pltpu.make_async_copy(
            o_v.at[0],
            out_hbm.at[pl.ds(0, BLK_ROWS), pl.ds(0, S_PER_W)],
            ssem,
        ).wait()

    def process(tb, b, first, refill):
        wait_gathers(b)
        if not first:
            wait_store()
        transpose_block(b)
        fire_store(tb, b)
        if refill:
            fire_gathers(tb + 2, b)

    fire_gathers(0, 0)
    fire_gathers(1, 1)
    process(0, 0, first=True, refill=True)
    process(1, 1, first=True, refill=True)

    def pair(o, _):
        for half in range(2):
            tb = 2 * o + half
            process(tb, half, first=False, refill=True)
        return ()

    lax.fori_loop(1, NBLK // 2 - 1, pair, ())

    process(NBLK - 2, 0, first=False, refill=False)
    process(NBLK - 1, 1, first=False, refill=False)
    wait_store()
    wait_store()


def kernel(input_ids, embed_tokens):
    ids2 = input_ids.T.astype(jnp.int32)
    out2d = _embed(ids2, embed_tokens)
    return out2d.reshape(T, HIDDEN, S).transpose(2, 0, 1)


# speedup vs baseline: 1.6500x; 1.6500x over previous
import functools

import jax
import jax.numpy as jnp
from jax import lax
from jax.experimental import pallas as pl
from jax.experimental.pallas import tpu as pltpu
from jax.experimental.pallas import tpu_sc as plsc

VOCAB = 1000000
HIDDEN = 64

NC = 2
NS = 16
NW = NC * NS

B_TOTAL = 4096 * 200
B_PER_W = B_TOTAL // NW
CHUNK = 400
N_CHUNKS = B_PER_W // CHUNK
NBUF = 4
K = 3
NQ = N_CHUNKS // NBUF

_mesh = plsc.VectorSubcoreMesh(core_axis_name="c", subcore_axis_name="s")


@functools.partial(
    pl.kernel,
    mesh=_mesh,
    compiler_params=pltpu.CompilerParams(use_tc_tiling_on_sc=False),
    out_type=jax.ShapeDtypeStruct((B_TOTAL // 200, 200, HIDDEN), jnp.float32),
    scratch_types=[
        pltpu.VMEM((B_PER_W,), jnp.int32),
        pltpu.VMEM((NBUF, 2, 200, HIDDEN), jnp.float32),
        pltpu.SemaphoreType.DMA,
        pltpu.SemaphoreType.DMA,
    ],
)
def _embed(idx_hbm, table_hbm, out_hbm, idx_v, rows_v, gsem, ssem):
    wid = lax.axis_index("s") * NC + lax.axis_index("c")
    base = pl.multiple_of(wid * B_PER_W, B_PER_W)
    pltpu.sync_copy(idx_hbm.at[pl.ds(base, B_PER_W)], idx_v)

    def fire_gather(j, b):
        for tj in range(2):
            off = pl.multiple_of(j * CHUNK + tj * 200, 8)
            pltpu.async_copy(
                table_hbm.at[idx_v.at[pl.ds(off, 200)]], rows_v.at[b, tj], gsem
            )

    def wait_gather(b):
        for tj in range(2):
            pltpu.make_async_copy(
                table_hbm.at[idx_v.at[pl.ds(0, 200)]], rows_v.at[b, tj], gsem
            ).wait()

    def fire_store(j, b):
        snt = pl.multiple_of((base + j * CHUNK) // 200, 2)
        pltpu.async_copy(rows_v.at[b], out_hbm.at[pl.ds(snt, 2)], ssem)

    def wait_store():
        pltpu.make_async_copy(
            rows_v.at[0], out_hbm.at[pl.ds(0, 2)], ssem
        ).wait()

    for j in range(K):
        fire_gather(j, j)

    for b in range(NBUF):
        if b >= 1:
            wait_store()
        wait_gather(b)
        fire_store(b, b)
        fire_gather(b + K, (b + K) % NBUF)

    def round_(o, _):
        for b in range(NBUF):
            i = o * NBUF + b
            wait_store()
            wait_gather(b)
            fire_store(i, b)
            fire_gather(i + K, (b + K) % NBUF)
        return ()

    lax.fori_loop(1, NQ - 1, round_, ())

    for b in range(NBUF):
        i = (NQ - 1) * NBUF + b
        wait_store()
        wait_gather(b)
        fire_store(i, b)
        if i + K < N_CHUNKS:
            fire_gather(i + K, (b + K) % NBUF)
    wait_store()


def kernel(input_ids, embed_tokens):
    flat = input_ids.reshape(-1).astype(jnp.int32)
    return _embed(flat, embed_tokens)


# speedup vs baseline: 1.6543x; 1.0026x over previous
import functools

import jax
import jax.numpy as jnp
from jax import lax
from jax.experimental import pallas as pl
from jax.experimental.pallas import tpu as pltpu
from jax.experimental.pallas import tpu_sc as plsc

VOCAB = 1000000
HIDDEN = 64

NC = 2
NS = 16
NW = NC * NS

B_TOTAL = 4096 * 200
B_PER_W = B_TOTAL // NW
CHUNK = 400
N_CHUNKS = B_PER_W // CHUNK
NBUF = 4
K = 3
NQ = N_CHUNKS // NBUF

_mesh = plsc.VectorSubcoreMesh(core_axis_name="c", subcore_axis_name="s")


@functools.partial(
    pl.kernel,
    mesh=_mesh,
    compiler_params=pltpu.CompilerParams(use_tc_tiling_on_sc=False),
    out_type=jax.ShapeDtypeStruct((B_TOTAL // 200, 200, HIDDEN), jnp.float32),
    scratch_types=[
        pltpu.VMEM((B_PER_W,), jnp.int32),
        pltpu.VMEM((NBUF, 2, 200, HIDDEN), jnp.float32),
        pltpu.SemaphoreType.DMA,
        pltpu.SemaphoreType.DMA,
    ],
)
def _embed(idx_hbm, table_hbm, out_hbm, idx_v, rows_v, gsem, ssem):
    wid = lax.axis_index("s") * NC + lax.axis_index("c")
    base = pl.multiple_of(wid * B_PER_W, B_PER_W)
    pltpu.sync_copy(idx_hbm.at[pl.ds(base, B_PER_W)], idx_v)

    def fire_gather(j, b):
        for tj in range(2):
            off = pl.multiple_of(j * CHUNK + tj * 200, 8)
            pltpu.async_copy(
                table_hbm.at[idx_v.at[pl.ds(off, 200)]], rows_v.at[b, tj], gsem
            )

    def wait_gather(b):
        for tj in range(2):
            pltpu.make_async_copy(
                table_hbm.at[idx_v.at[pl.ds(0, 200)]], rows_v.at[b, tj], gsem
            ).wait()

    def fire_store(j, b):
        snt = pl.multiple_of((base + j * CHUNK) // 200, 2)
        pltpu.async_copy(rows_v.at[b], out_hbm.at[pl.ds(snt, 2)], ssem)

    def wait_store():
        pltpu.make_async_copy(
            rows_v.at[0], out_hbm.at[pl.ds(0, 2)], ssem
        ).wait()

    for j in range(K):
        fire_gather(j, j)

    for b in range(NBUF):
        if b >= 1:
            wait_store()
        wait_gather(b)
        fire_store(b, b)
        fire_gather(b + K, (b + K) % NBUF)

    def round_(o, _):
        for b in range(NBUF):
            i = o * NBUF + b
            wait_store()
            wait_gather(b)
            fire_store(i, b)
            fire_gather(i + K, (b + K) % NBUF)
        return ()

    lax.fori_loop(1, NQ - 1, round_, ())

    for b in range(NBUF):
        i = (NQ - 1) * NBUF + b
        wait_store()
        wait_gather(b)
        fire_store(i, b)
        if i + K < N_CHUNKS:
            fire_gather(i + K, (b + K) % NBUF)
    wait_store()


def kernel(input_ids, embed_tokens):
    flat = input_ids.reshape(-1).astype(jnp.int32)
    one = (flat[0] * 0 + 1).astype(jnp.float32)
    return _embed(flat, embed_tokens * one)


# speedup vs baseline: 2.0096x; 1.2148x over previous
import functools

import jax
import jax.numpy as jnp
from jax import lax
from jax.experimental import pallas as pl
from jax.experimental.pallas import tpu as pltpu
from jax.experimental.pallas import tpu_sc as plsc

VOCAB = 1000000
HIDDEN = 64
HPAD = 128

NC = 2
NS = 16
NW = NC * NS

B_TOTAL = 4096 * 200
B_PER_W = B_TOTAL // NW
CHUNK = 400
N_CHUNKS = B_PER_W // CHUNK
NBUF = 2
K = 1
NQ = N_CHUNKS // NBUF

_mesh = plsc.VectorSubcoreMesh(core_axis_name="c", subcore_axis_name="s")


@functools.partial(
    pl.kernel,
    mesh=_mesh,
    compiler_params=pltpu.CompilerParams(use_tc_tiling_on_sc=False),
    out_type=jax.ShapeDtypeStruct((B_TOTAL // 200, 200, HPAD), jnp.float32),
    scratch_types=[
        pltpu.VMEM((B_PER_W,), jnp.int32),
        pltpu.VMEM((NBUF, 2, 200, HPAD), jnp.float32),
        pltpu.SemaphoreType.DMA,
        pltpu.SemaphoreType.DMA,
    ],
)
def _embed(idx_hbm, table_hbm, out_hbm, idx_v, rows_v, gsem, ssem):
    wid = lax.axis_index("s") * NC + lax.axis_index("c")
    base = pl.multiple_of(wid * B_PER_W, B_PER_W)
    pltpu.sync_copy(idx_hbm.at[pl.ds(base, B_PER_W)], idx_v)

    def fire_gather(j, b):
        for tj in range(2):
            off = pl.multiple_of(j * CHUNK + tj * 200, 8)
            pltpu.async_copy(
                table_hbm.at[idx_v.at[pl.ds(off, 200)]], rows_v.at[b, tj], gsem
            )

    def wait_gather(b):
        for tj in range(2):
            pltpu.make_async_copy(
                table_hbm.at[idx_v.at[pl.ds(0, 200)]], rows_v.at[b, tj], gsem
            ).wait()

    def fire_store(j, b):
        snt = pl.multiple_of((base + j * CHUNK) // 200, 2)
        pltpu.async_copy(rows_v.at[b], out_hbm.at[pl.ds(snt, 2)], ssem)

    def wait_store():
        pltpu.make_async_copy(
            rows_v.at[0], out_hbm.at[pl.ds(0, 2)], ssem
        ).wait()

    for j in range(K):
        fire_gather(j, j)

    for b in range(NBUF):
        if b >= 1:
            wait_store()
        wait_gather(b)
        fire_store(b, b)
        fire_gather(b + K, (b + K) % NBUF)

    def round_(o, _):
        for b in range(NBUF):
            i = o * NBUF + b
            wait_store()
            wait_gather(b)
            fire_store(i, b)
            fire_gather(i + K, (b + K) % NBUF)
        return ()

    lax.fori_loop(1, NQ - 1, round_, ())

    for b in range(NBUF):
        i = (NQ - 1) * NBUF + b
        wait_store()
        wait_gather(b)
        fire_store(i, b)
        if i + K < N_CHUNKS:
            fire_gather(i + K, (b + K) % NBUF)
    wait_store()


def kernel(input_ids, embed_tokens):
    flat = input_ids.reshape(-1).astype(jnp.int32)
    padded = jnp.pad(embed_tokens, ((0, 0), (0, HPAD - HIDDEN)))
    out = _embed(flat, padded)
    return out[:, :, :HIDDEN]


# speedup vs baseline: 2.3473x; 1.1680x over previous
import functools

import jax
import jax.numpy as jnp
from jax import lax
from jax.experimental import pallas as pl
from jax.experimental.pallas import tpu as pltpu
from jax.experimental.pallas import tpu_sc as plsc

VOCAB = 1000000
HIDDEN = 64
HPAD = 128

NC = 2
NS = 16
NW = NC * NS

B_TOTAL = 4096 * 200
B_PER_W = B_TOTAL // NW
CHUNK = 400
N_CHUNKS = B_PER_W // CHUNK
NBUF = 2
K = 1
NQ = N_CHUNKS // NBUF

_mesh = plsc.VectorSubcoreMesh(core_axis_name="c", subcore_axis_name="s")


@functools.partial(
    pl.kernel,
    mesh=_mesh,
    compiler_params=pltpu.CompilerParams(use_tc_tiling_on_sc=False),
    out_type=jax.ShapeDtypeStruct((B_TOTAL // 200, 200, HPAD), jnp.float32),
    scratch_types=[
        pltpu.VMEM((B_PER_W,), jnp.int32),
        pltpu.VMEM((NBUF, 2, 200, HIDDEN), jnp.float32),
        pltpu.SemaphoreType.DMA,
        pltpu.SemaphoreType.DMA,
    ],
)
def _embed(idx_hbm, table_hbm, out_hbm, idx_v, rows_v, gsem, ssem):
    wid = lax.axis_index("s") * NC + lax.axis_index("c")
    base = pl.multiple_of(wid * B_PER_W, B_PER_W)
    pltpu.sync_copy(idx_hbm.at[pl.ds(base, B_PER_W)], idx_v)

    def fire_gather(j, b):
        for tj in range(2):
            off = pl.multiple_of(j * CHUNK + tj * 200, 8)
            pltpu.async_copy(
                table_hbm.at[idx_v.at[pl.ds(off, 200)]], rows_v.at[b, tj], gsem
            )

    def wait_gather(b):
        for tj in range(2):
            pltpu.make_async_copy(
                table_hbm.at[idx_v.at[pl.ds(0, 200)]], rows_v.at[b, tj], gsem
            ).wait()

    def fire_store(j, b):
        snt = pl.multiple_of((base + j * CHUNK) // 200, 2)
        pltpu.async_copy(
            rows_v.at[b], out_hbm.at[pl.ds(snt, 2), :, pl.ds(0, HIDDEN)], ssem
        )

    def wait_store():
        pltpu.make_async_copy(
            rows_v.at[0], out_hbm.at[pl.ds(0, 2), :, pl.ds(0, HIDDEN)], ssem
        ).wait()

    for j in range(K):
        fire_gather(j, j)

    for b in range(NBUF):
        if b >= 1:
            wait_store()
        wait_gather(b)
        fire_store(b, b)
        fire_gather(b + K, (b + K) % NBUF)

    def round_(o, _):
        for b in range(NBUF):
            i = o * NBUF + b
            wait_store()
            wait_gather(b)
            fire_store(i, b)
            fire_gather(i + K, (b + K) % NBUF)
        return ()

    lax.fori_loop(1, NQ - 1, round_, ())

    for b in range(NBUF):
        i = (NQ - 1) * NBUF + b
        wait_store()
        wait_gather(b)
        fire_store(i, b)
        if i + K < N_CHUNKS:
            fire_gather(i + K, (b + K) % NBUF)
    wait_store()


def kernel(input_ids, embed_tokens):
    flat = input_ids.reshape(-1).astype(jnp.int32) * 2
    padded = jnp.pad(embed_tokens, ((0, 0), (0, HPAD - HIDDEN)))
    out = _embed(flat, padded.reshape(2 * VOCAB, HIDDEN))
    return out[:, :, :HIDDEN]


# speedup vs baseline: 2.3611x; 1.0059x over previous
import functools

import jax
import jax.numpy as jnp
from jax import lax
from jax.experimental import pallas as pl
from jax.experimental.pallas import tpu as pltpu
from jax.experimental.pallas import tpu_sc as plsc

VOCAB = 1000000
HIDDEN = 64
HPAD = 128

NC = 2
NS = 16
NW = NC * NS

B_TOTAL = 4096 * 200
B_PER_W = B_TOTAL // NW
CHUNK = 400
N_CHUNKS = B_PER_W // CHUNK
NBUF = 4
K = 3
NQ = N_CHUNKS // NBUF

_mesh = plsc.VectorSubcoreMesh(core_axis_name="c", subcore_axis_name="s")


@functools.partial(
    pl.kernel,
    mesh=_mesh,
    compiler_params=pltpu.CompilerParams(use_tc_tiling_on_sc=False),
    out_type=jax.ShapeDtypeStruct((B_TOTAL // 200, 200, HPAD), jnp.float32),
    scratch_types=[
        pltpu.VMEM((B_PER_W,), jnp.int32),
        pltpu.VMEM((NBUF, 2, 200, HIDDEN), jnp.float32),
        pltpu.SemaphoreType.DMA,
        pltpu.SemaphoreType.DMA,
    ],
)
def _embed(idx_hbm, table_hbm, out_hbm, idx_v, rows_v, gsem, ssem):
    wid = lax.axis_index("s") * NC + lax.axis_index("c")
    base = pl.multiple_of(wid * B_PER_W, B_PER_W)
    pltpu.sync_copy(idx_hbm.at[pl.ds(base, B_PER_W)], idx_v)

    def fire_gather(j, b):
        for tj in range(2):
            off = pl.multiple_of(j * CHUNK + tj * 200, 8)
            pltpu.async_copy(
                table_hbm.at[idx_v.at[pl.ds(off, 200)]], rows_v.at[b, tj], gsem
            )

    def wait_gather(b):
        for tj in range(2):
            pltpu.make_async_copy(
                table_hbm.at[idx_v.at[pl.ds(0, 200)]], rows_v.at[b, tj], gsem
            ).wait()

    def fire_store(j, b):
        snt = pl.multiple_of((base + j * CHUNK) // 200, 2)
        pltpu.async_copy(
            rows_v.at[b], out_hbm.at[pl.ds(snt, 2), :, pl.ds(0, HIDDEN)], ssem
        )

    def wait_store():
        pltpu.make_async_copy(
            rows_v.at[0], out_hbm.at[pl.ds(0, 2), :, pl.ds(0, HIDDEN)], ssem
        ).wait()

    for j in range(K):
        fire_gather(j, j)

    for b in range(NBUF):
        if b >= 1:
            wait_store()
        wait_gather(b)
        fire_store(b, b)
        fire_gather(b + K, (b + K) % NBUF)

    def round_(o, _):
        for b in range(NBUF):
            i = o * NBUF + b
            wait_store()
            wait_gather(b)
            fire_store(i, b)
            fire_gather(i + K, (b + K) % NBUF)
        return ()

    lax.fori_loop(1, NQ - 1, round_, ())

    for b in range(NBUF):
        i = (NQ - 1) * NBUF + b
        wait_store()
        wait_gather(b)
        fire_store(i, b)
        if i + K < N_CHUNKS:
            fire_gather(i + K, (b + K) % NBUF)
    wait_store()


def kernel(input_ids, embed_tokens):
    flat = input_ids.reshape(-1).astype(jnp.int32) * 2
    padded = jnp.pad(embed_tokens, ((0, 0), (0, HPAD - HIDDEN)))
    out = _embed(flat, padded.reshape(2 * VOCAB, HIDDEN))
    return out[:, :, :HIDDEN]
